# baseline (speedup 1.0000x reference)
import functools

import jax
import jax.numpy as jnp
from jax import lax
from jax.experimental import pallas as pl
from jax.experimental.pallas import tpu as pltpu
from jax.experimental.pallas import tpu_sc as plsc

N = 10000
NPAD = 10240
D = 128
E = 320000
NC, NS = 2, 16
NW = NC * NS
EPT = E // NW
K = 64
EPTP = 10240
NCHUNK = EPTP // K
GRP = 8
NGRP = NCHUNK // GRP
RPT = NPAD // NS

_F32 = jnp.float32


def _sc_mesh():
    return plsc.VectorSubcoreMesh(
        core_axis_name="c", subcore_axis_name="s", num_cores=NC, num_subcores=NS
    )


def _sc_deg_body(dstc_hbm, ones_hbm, zero_hbm, out_hbm, dst_v, ones_v, deg_sh):
    cid = lax.axis_index("c")
    sid = lax.axis_index("s")
    wid = sid * NC + cid
    pltpu.sync_copy(zero_hbm.at[pl.ds(sid * RPT, RPT)],
                    deg_sh.at[pl.ds(sid * RPT, RPT)])
    pltpu.sync_copy(dstc_hbm.at[wid], dst_v)
    pltpu.sync_copy(ones_hbm, ones_v)
    plsc.subcore_barrier()

    def chunk(j, carry):
        pltpu.sync_copy(ones_v, deg_sh.at[dst_v.at[j]], add=True)
        return carry

    lax.fori_loop(0, NCHUNK, chunk, 0, unroll=False)
    plsc.subcore_barrier()
    pltpu.sync_copy(deg_sh.at[pl.ds(sid * RPT, RPT)],
                    out_hbm.at[cid, pl.ds(sid * RPT, RPT)])


def _sc_degree(dst3):
    ones = jnp.ones((K, D), _F32)
    zeros = jnp.zeros((NPAD, D), _F32)
    return pl.kernel(
        _sc_deg_body,
        out_type=jax.ShapeDtypeStruct((NC, NPAD, D), _F32),
        mesh=_sc_mesh(),
        scratch_types=[
            pltpu.VMEM((NCHUNK, K), jnp.int32),
            pltpu.VMEM((K, D), _F32),
            pltpu.VMEM_SHARED((NPAD, D), _F32),
        ],
    )(dst3, ones, zeros)


def _sc_agg_body(y_hbm, srcc_hbm, dstc_hbm, zero_hbm, out_hbm,
                 src_v, dstg_v, rows0_v, rows1_v, z_sh, sem0, sem1):
    cid = lax.axis_index("c")
    sid = lax.axis_index("s")
    wid = sid * NC + cid
    pltpu.sync_copy(zero_hbm.at[pl.ds(sid * RPT, RPT)],
                    z_sh.at[pl.ds(sid * RPT, RPT)])
    pltpu.sync_copy(srcc_hbm.at[wid], src_v)
    plsc.subcore_barrier()

    rows = (rows0_v, rows1_v)
    sems = (sem0, sem1)

    def gather(j, buf, sem):
        pltpu.async_copy(y_hbm.at[src_v.at[j]], buf, sem)

    def chunk(b, i):
        j = i * GRP + b
        pltpu.async_copy(y_hbm.at[src_v.at[j]], rows0_v, sem0).wait()
        pltpu.sync_copy(rows0_v, z_sh.at[dstg_v.at[b]], add=True)
        return i

    def group(i, carry):
        pltpu.sync_copy(dstc_hbm.at[wid, pl.ds(i * GRP, GRP)], dstg_v)
        lax.fori_loop(0, GRP, chunk, i, unroll=False)
        return carry

    lax.fori_loop(0, NGRP, group, 0, unroll=False)
    plsc.subcore_barrier()
    pltpu.sync_copy(z_sh.at[pl.ds(sid * RPT, RPT)],
                    out_hbm.at[cid, pl.ds(sid * RPT, RPT)])


def _sc_aggregate(y, src3, dst3):
    zeros = jnp.zeros((NPAD, D), _F32)
    return pl.kernel(
        _sc_agg_body,
        out_type=jax.ShapeDtypeStruct((NC, NPAD, D), _F32),
        mesh=_sc_mesh(),
        scratch_types=[
            pltpu.VMEM((NCHUNK, K), jnp.int32),
            pltpu.VMEM((GRP, K), jnp.int32),
            pltpu.VMEM((K, D), _F32),
            pltpu.VMEM((K, D), _F32),
            pltpu.VMEM_SHARED((NPAD, D), _F32),
            pltpu.SemaphoreType.DMA,
            pltpu.SemaphoreType.DMA,
        ],
    )(y, src3, dst3, zeros)


_R = 2000
_G = N // _R

def _row_spec(w):
    return pl.BlockSpec((_R, w), lambda i: (i, 0))

def _z_spec(w):
    return pl.BlockSpec((NC, _R, w), lambda i: (0, i, 0))

def _rep_spec(shape):
    return pl.BlockSpec(shape, lambda i: tuple(0 for _ in shape))


def _tc_pre_body(degp_ref, x_ref, w_ref, dis_ref, xw_ref, y_ref):
    deg = degp_ref[0, :, :1] + degp_ref[1, :, :1] + 1.0
    dis = lax.rsqrt(deg)
    xw = jnp.dot(x_ref[...], w_ref[...], preferred_element_type=_F32)
    dis_ref[...] = dis
    xw_ref[...] = xw
    y_ref[...] = xw * dis


def _tc_pre(degp, x, w):
    return pl.pallas_call(
        _tc_pre_body,
        grid=(_G,),
        in_specs=[_z_spec(D), _row_spec(D), _rep_spec((D, D))],
        out_specs=[_row_spec(1), _row_spec(D), _row_spec(D)],
        out_shape=[
            jax.ShapeDtypeStruct((N, 1), _F32),
            jax.ShapeDtypeStruct((N, D), _F32),
            jax.ShapeDtypeStruct((N, D), _F32),
        ],
    )(degp, x, w)


def _tc_mid_body(dis_ref, xw_ref, z_ref, b_ref, hprev_ref, w_ref,
                 h_ref, xwn_ref, yn_ref, *, residual):
    dis = dis_ref[...]
    z = z_ref[0] + z_ref[1]
    out = dis * z + (dis * dis) * xw_ref[...] + b_ref[...]
    h = jnp.maximum(out, 0.0)
    if residual:
        h = h + hprev_ref[...]
    xwn = jnp.dot(h, w_ref[...], preferred_element_type=_F32)
    h_ref[...] = h
    xwn_ref[...] = xwn
    yn_ref[...] = xwn * dis


def _tc_mid(dis, xw, z, b, hprev, w, residual):
    return pl.pallas_call(
        functools.partial(_tc_mid_body, residual=residual),
        grid=(_G,),
        in_specs=[_row_spec(1), _row_spec(D), _z_spec(D),
                  _rep_spec((1, D)), _row_spec(D), _rep_spec((D, D))],
        out_specs=[_row_spec(D), _row_spec(D), _row_spec(D)],
        out_shape=[
            jax.ShapeDtypeStruct((N, D), _F32),
            jax.ShapeDtypeStruct((N, D), _F32),
            jax.ShapeDtypeStruct((N, D), _F32),
        ],
    )(dis, xw, z, b, hprev, w)


def _bn(t, g, be, rm, rv):
    return (t - rm) * lax.rsqrt(rv + 1e-5) * g + be


def _tc_fin_body(dis_ref, xw_ref, z_ref, b_ref, hprev_ref,
                 rW1_ref, rb1_ref, rg1_ref, rbe1_ref, rrm1_ref, rrv1_ref,
                 rW2t_ref, rb2_ref,
                 pW1_ref, pb1_ref, pg1_ref, pbe1_ref, prm1_ref, prv1_ref,
                 pW2_ref, pb2_ref, pg2_ref, pbe2_ref, prm2_ref, prv2_ref,
                 pW3_ref, pb3_ref, out_ref):
    dis = dis_ref[...]
    z = z_ref[0] + z_ref[1]
    h3 = jnp.maximum(dis * z + (dis * dis) * xw_ref[...] + b_ref[...], 0.0)
    h3 = h3 + hprev_ref[...]

    r = jnp.dot(h3, rW1_ref[...], preferred_element_type=_F32) + rb1_ref[...]
    r = jnp.maximum(_bn(r, rg1_ref[...], rbe1_ref[...], rrm1_ref[...],
                        rrv1_ref[...]), 0.0)
    rad_lin = jnp.sum(r * rW2t_ref[...], axis=1, keepdims=True) + rb2_ref[...]
    radius = jnp.maximum(rad_lin, 0.0) + jnp.log1p(jnp.exp(-jnp.abs(rad_lin)))

    p = jnp.dot(h3, pW1_ref[...], preferred_element_type=_F32) + pb1_ref[...]
    p = jnp.maximum(_bn(p, pg1_ref[...], pbe1_ref[...], prm1_ref[...],
                        prv1_ref[...]), 0.0)
    p = jnp.dot(p, pW2_ref[...], preferred_element_type=_F32) + pb2_ref[...]
    p = jnp.maximum(_bn(p, pg2_ref[...], pbe2_ref[...], prm2_ref[...],
                        prv2_ref[...]), 0.0)
    coords = jnp.dot(p, pW3_ref[...], preferred_element_type=_F32) + pb3_ref[...]
    nrm = jnp.maximum(jnp.sqrt(jnp.sum(coords * coords, axis=1, keepdims=True)),
                      1e-12)
    out_ref[...] = coords / nrm * radius


def _tc_fin(dis, xw, z, b, hprev, rW1, rb1, rg1, rbe1, rrm1, rrv1,
            rW2t, rb2, pW1, pb1, pg1, pbe1, prm1, prv1, pW2, pb2, pg2, pbe2,
            prm2, prv2, pW3p, pb3p):
    H2 = D // 2
    return pl.pallas_call(
        _tc_fin_body,
        grid=(_G,),
        in_specs=[
            _row_spec(1), _row_spec(D), _z_spec(D),
            _rep_spec((1, D)), _row_spec(D),
            _rep_spec((D, H2)), _rep_spec((1, H2)), _rep_spec((1, H2)),
            _rep_spec((1, H2)), _rep_spec((1, H2)), _rep_spec((1, H2)),
            _rep_spec((1, H2)), _rep_spec((1, 1)),
            _rep_spec((D, D)), _rep_spec((1, D)), _rep_spec((1, D)),
            _rep_spec((1, D)), _rep_spec((1, D)), _rep_spec((1, D)),
            _rep_spec((D, H2)), _rep_spec((1, H2)), _rep_spec((1, H2)),
            _rep_spec((1, H2)), _rep_spec((1, H2)), _rep_spec((1, H2)),
            _rep_spec((H2, D)), _rep_spec((1, D)),
        ],
        out_specs=[_row_spec(D)],
        out_shape=[jax.ShapeDtypeStruct((N, D), _F32)],
    )(dis, xw, z, b, hprev, rW1, rb1, rg1, rbe1, rrm1, rrv1, rW2t, rb2,
      pW1, pb1, pg1, pbe1, prm1, prv1, pW2, pb2, pg2, pbe2, prm2, prv2,
      pW3p, pb3p)[0]


def kernel(x, edge_index, c1_W, c1_b, c2_W, c2_b, c3_W, c3_b,
           p_W1, p_b1, p_g1, p_be1, p_rm1, p_rv1,
           p_W2, p_b2, p_g2, p_be2, p_rm2, p_rv2, p_W3, p_b3,
           r_W1, r_b1, r_g1, r_be1, r_rm1, r_rv1, r_W2, r_b2):
    pad = EPTP - EPT
    pad_dst = jnp.broadcast_to(N + jnp.arange(pad, dtype=jnp.int32), (NW, pad))
    src3 = jnp.pad(edge_index[0].reshape(NW, EPT), ((0, 0), (0, pad))
                   ).reshape(NW, NCHUNK, K)
    dst3 = jnp.concatenate([edge_index[1].reshape(NW, EPT), pad_dst], axis=1
                           ).reshape(NW, NCHUNK, K)

    degp = _sc_degree(dst3)
    dis, xw1, y1 = _tc_pre(degp, x, c1_W)

    z1 = _sc_aggregate(y1, src3, dst3)
    h1, xw2, y2 = _tc_mid(dis, xw1, z1, c1_b.reshape(1, D), x, c2_W,
                          residual=False)

    z2 = _sc_aggregate(y2, src3, dst3)
    h2, xw3, y3 = _tc_mid(dis, xw2, z2, c2_b.reshape(1, D), h1, c3_W,
                          residual=True)

    z3 = _sc_aggregate(y3, src3, dst3)

    H2 = D // 2
    pW3p = jnp.zeros((H2, D), _F32).at[:, :2].set(p_W3)
    pb3p = jnp.zeros((1, D), _F32).at[:, :2].set(p_b3.reshape(1, 2))
    out = _tc_fin(
        dis, xw3, z3, c3_b.reshape(1, D), h2,
        r_W1, r_b1.reshape(1, H2), r_g1.reshape(1, H2), r_be1.reshape(1, H2),
        r_rm1.reshape(1, H2), r_rv1.reshape(1, H2),
        r_W2.reshape(1, H2), r_b2.reshape(1, 1),
        p_W1, p_b1.reshape(1, D), p_g1.reshape(1, D), p_be1.reshape(1, D),
        p_rm1.reshape(1, D), p_rv1.reshape(1, D),
        p_W2, p_b2.reshape(1, H2), p_g2.reshape(1, H2), p_be2.reshape(1, H2),
        p_rm2.reshape(1, H2), p_rv2.reshape(1, H2),
        pW3p, pb3p)
    return out[:, :2]


# speedup vs baseline: 2.1933x; 2.1933x over previous
import functools

import jax
import jax.numpy as jnp
from jax import lax
from jax.experimental import pallas as pl
from jax.experimental.pallas import tpu as pltpu
from jax.experimental.pallas import tpu_sc as plsc

N = 10000
NPAD = 10240
D = 128
E = 320000
NC, NS = 2, 16
NW = NC * NS
EPT = E // NW
K = 80
NCHUNK = EPT // K
RPT = NPAD // NS

_F32 = jnp.float32


def _sc_mesh():
    return plsc.VectorSubcoreMesh(
        core_axis_name="c", subcore_axis_name="s", num_cores=NC, num_subcores=NS
    )


def _sc_deg_body(dstc_hbm, ones_hbm, zero_hbm, out_hbm, dst_v, ones_v, deg_sh,
                 sem):
    cid = lax.axis_index("c")
    sid = lax.axis_index("s")
    wid = sid * NC + cid
    pltpu.sync_copy(zero_hbm.at[pl.ds(sid * RPT, RPT)],
                    deg_sh.at[pl.ds(sid * RPT, RPT)])
    pltpu.sync_copy(dstc_hbm.at[wid], dst_v)
    pltpu.sync_copy(ones_hbm, ones_v)
    plsc.subcore_barrier()

    def chunk(j, carry):
        pltpu.async_copy(ones_v, deg_sh.at[dst_v.at[j]], sem, add=True)
        return carry

    lax.fori_loop(0, NCHUNK, chunk, 0, unroll=False)

    def drain(j, carry):
        pltpu.make_async_copy(ones_v, deg_sh.at[dst_v.at[j]], sem).wait()
        return carry

    lax.fori_loop(0, NCHUNK, drain, 0, unroll=False)
    plsc.subcore_barrier()
    pltpu.sync_copy(deg_sh.at[pl.ds(sid * RPT, RPT)],
                    out_hbm.at[cid, pl.ds(sid * RPT, RPT)])


def _sc_degree(dst3):
    ones = jnp.ones((K, D), _F32)
    zeros = jnp.zeros((NPAD, D), _F32)
    return pl.kernel(
        _sc_deg_body,
        out_type=jax.ShapeDtypeStruct((NC, NPAD, D), _F32),
        mesh=_sc_mesh(),
        scratch_types=[
            pltpu.VMEM((NCHUNK, K), jnp.int32),
            pltpu.VMEM((K, D), _F32),
            pltpu.VMEM_SHARED((NPAD, D), _F32),
            pltpu.SemaphoreType.DMA,
        ],
    )(dst3, ones, zeros)


def _sc_agg_body(y_hbm, srcc_hbm, dstc_hbm, zero_hbm, out_hbm,
                 src_v, dst_v, rows_v, z_sh, sem):
    cid = lax.axis_index("c")
    sid = lax.axis_index("s")
    wid = sid * NC + cid
    pltpu.sync_copy(zero_hbm.at[pl.ds(sid * RPT, RPT)],
                    z_sh.at[pl.ds(sid * RPT, RPT)])
    pltpu.sync_copy(srcc_hbm.at[wid], src_v)
    pltpu.sync_copy(dstc_hbm.at[wid], dst_v)
    plsc.subcore_barrier()

    def chunk(j, carry):
        pltpu.async_copy(y_hbm.at[src_v.at[j]], rows_v, sem).wait()
        pltpu.sync_copy(rows_v, z_sh.at[dst_v.at[j]], add=True)
        return carry

    lax.fori_loop(0, NCHUNK, chunk, 0, unroll=False)
    plsc.subcore_barrier()
    pltpu.sync_copy(z_sh.at[pl.ds(sid * RPT, RPT)],
                    out_hbm.at[cid, pl.ds(sid * RPT, RPT)])


def _sc_aggregate(y, src3, dst3):
    zeros = jnp.zeros((NPAD, D), _F32)
    return pl.kernel(
        _sc_agg_body,
        out_type=jax.ShapeDtypeStruct((NC, NPAD, D), _F32),
        mesh=_sc_mesh(),
        scratch_types=[
            pltpu.VMEM((NCHUNK, K), jnp.int32),
            pltpu.VMEM((NCHUNK, K), jnp.int32),
            pltpu.VMEM((K, D), _F32),
            pltpu.VMEM_SHARED((NPAD, D), _F32),
            pltpu.SemaphoreType.DMA,
        ],
    )(y, src3, dst3, zeros)


_R = 2000
_G = N // _R

def _row_spec(w):
    return pl.BlockSpec((_R, w), lambda i: (i, 0))

def _z_spec(w):
    return pl.BlockSpec((NC, _R, w), lambda i: (0, i, 0))

def _rep_spec(shape):
    return pl.BlockSpec(shape, lambda i: tuple(0 for _ in shape))


def _tc_mm_body(x_ref, w_ref, xw_ref):
    xw_ref[...] = jnp.dot(x_ref[...], w_ref[...], preferred_element_type=_F32)


def _tc_mm(x, w):
    return pl.pallas_call(
        _tc_mm_body,
        grid=(_G,),
        in_specs=[_row_spec(D), _rep_spec((D, D))],
        out_specs=[_row_spec(D)],
        out_shape=[jax.ShapeDtypeStruct((N, D), _F32)],
    )(x, w)[0]


def _tc_pre_body(degp_ref, xw_ref, dis_ref, y_ref):
    deg = degp_ref[0, :, :1] + degp_ref[1, :, :1] + 1.0
    dis = lax.rsqrt(deg)
    dis_ref[...] = dis
    y_ref[...] = xw_ref[...] * dis


def _tc_pre(degp, xw):
    return pl.pallas_call(
        _tc_pre_body,
        grid=(_G,),
        in_specs=[_z_spec(D), _row_spec(D)],
        out_specs=[_row_spec(1), _row_spec(D)],
        out_shape=[
            jax.ShapeDtypeStruct((N, 1), _F32),
            jax.ShapeDtypeStruct((N, D), _F32),
        ],
    )(degp, xw)


def _tc_mid_body(dis_ref, xw_ref, z_ref, b_ref, hprev_ref, w_ref,
                 h_ref, xwn_ref, yn_ref, *, residual):
    dis = dis_ref[...]
    z = z_ref[0] + z_ref[1]
    out = dis * z + (dis * dis) * xw_ref[...] + b_ref[...]
    h = jnp.maximum(out, 0.0)
    if residual:
        h = h + hprev_ref[...]
    xwn = jnp.dot(h, w_ref[...], preferred_element_type=_F32)
    h_ref[...] = h
    xwn_ref[...] = xwn
    yn_ref[...] = xwn * dis


def _tc_mid(dis, xw, z, b, hprev, w, residual):
    return pl.pallas_call(
        functools.partial(_tc_mid_body, residual=residual),
        grid=(_G,),
        in_specs=[_row_spec(1), _row_spec(D), _z_spec(D),
                  _rep_spec((1, D)), _row_spec(D), _rep_spec((D, D))],
        out_specs=[_row_spec(D), _row_spec(D), _row_spec(D)],
        out_shape=[
            jax.ShapeDtypeStruct((N, D), _F32),
            jax.ShapeDtypeStruct((N, D), _F32),
            jax.ShapeDtypeStruct((N, D), _F32),
        ],
    )(dis, xw, z, b, hprev, w)


def _bn(t, g, be, rm, rv):
    return (t - rm) * lax.rsqrt(rv + 1e-5) * g + be


def _tc_fin_body(dis_ref, xw_ref, z_ref, b_ref, hprev_ref,
                 rW1_ref, rb1_ref, rg1_ref, rbe1_ref, rrm1_ref, rrv1_ref,
                 rW2t_ref, rb2_ref,
                 pW1_ref, pb1_ref, pg1_ref, pbe1_ref, prm1_ref, prv1_ref,
                 pW2_ref, pb2_ref, pg2_ref, pbe2_ref, prm2_ref, prv2_ref,
                 pW3_ref, pb3_ref, out_ref):
    dis = dis_ref[...]
    z = z_ref[0] + z_ref[1]
    h3 = jnp.maximum(dis * z + (dis * dis) * xw_ref[...] + b_ref[...], 0.0)
    h3 = h3 + hprev_ref[...]

    r = jnp.dot(h3, rW1_ref[...], preferred_element_type=_F32) + rb1_ref[...]
    r = jnp.maximum(_bn(r, rg1_ref[...], rbe1_ref[...], rrm1_ref[...],
                        rrv1_ref[...]), 0.0)
    rad_lin = jnp.sum(r * rW2t_ref[...], axis=1, keepdims=True) + rb2_ref[...]
    radius = jnp.maximum(rad_lin, 0.0) + jnp.log1p(jnp.exp(-jnp.abs(rad_lin)))

    p = jnp.dot(h3, pW1_ref[...], preferred_element_type=_F32) + pb1_ref[...]
    p = jnp.maximum(_bn(p, pg1_ref[...], pbe1_ref[...], prm1_ref[...],
                        prv1_ref[...]), 0.0)
    p = jnp.dot(p, pW2_ref[...], preferred_element_type=_F32) + pb2_ref[...]
    p = jnp.maximum(_bn(p, pg2_ref[...], pbe2_ref[...], prm2_ref[...],
                        prv2_ref[...]), 0.0)
    coords = jnp.dot(p, pW3_ref[...], preferred_element_type=_F32) + pb3_ref[...]
    nrm = jnp.maximum(jnp.sqrt(jnp.sum(coords * coords, axis=1, keepdims=True)),
                      1e-12)
    out_ref[...] = coords / nrm * radius


def _tc_fin(dis, xw, z, b, hprev, rW1, rb1, rg1, rbe1, rrm1, rrv1,
            rW2t, rb2, pW1, pb1, pg1, pbe1, prm1, prv1, pW2, pb2, pg2, pbe2,
            prm2, prv2, pW3p, pb3p):
    H2 = D // 2
    return pl.pallas_call(
        _tc_fin_body,
        grid=(_G,),
        in_specs=[
            _row_spec(1), _row_spec(D), _z_spec(D),
            _rep_spec((1, D)), _row_spec(D),
            _rep_spec((D, H2)), _rep_spec((1, H2)), _rep_spec((1, H2)),
            _rep_spec((1, H2)), _rep_spec((1, H2)), _rep_spec((1, H2)),
            _rep_spec((1, H2)), _rep_spec((1, 1)),
            _rep_spec((D, D)), _rep_spec((1, D)), _rep_spec((1, D)),
            _rep_spec((1, D)), _rep_spec((1, D)), _rep_spec((1, D)),
            _rep_spec((D, H2)), _rep_spec((1, H2)), _rep_spec((1, H2)),
            _rep_spec((1, H2)), _rep_spec((1, H2)), _rep_spec((1, H2)),
            _rep_spec((H2, D)), _rep_spec((1, D)),
        ],
        out_specs=[_row_spec(D)],
        out_shape=[jax.ShapeDtypeStruct((N, D), _F32)],
    )(dis, xw, z, b, hprev, rW1, rb1, rg1, rbe1, rrm1, rrv1, rW2t, rb2,
      pW1, pb1, pg1, pbe1, prm1, prv1, pW2, pb2, pg2, pbe2, prm2, prv2,
      pW3p, pb3p)[0]


def kernel(x, edge_index, c1_W, c1_b, c2_W, c2_b, c3_W, c3_b,
           p_W1, p_b1, p_g1, p_be1, p_rm1, p_rv1,
           p_W2, p_b2, p_g2, p_be2, p_rm2, p_rv2, p_W3, p_b3,
           r_W1, r_b1, r_g1, r_be1, r_rm1, r_rv1, r_W2, r_b2):
    src3 = edge_index[0].reshape(NW, NCHUNK, K)
    dst3 = edge_index[1].reshape(NW, NCHUNK, K)

    degp = _sc_degree(dst3)
    xw1 = _tc_mm(x, c1_W)
    dis, y1 = _tc_pre(degp, xw1)

    z1 = _sc_aggregate(y1, src3, dst3)
    h1, xw2, y2 = _tc_mid(dis, xw1, z1, c1_b.reshape(1, D), x, c2_W,
                          residual=False)

    z2 = _sc_aggregate(y2, src3, dst3)
    h2, xw3, y3 = _tc_mid(dis, xw2, z2, c2_b.reshape(1, D), h1, c3_W,
                          residual=True)

    z3 = _sc_aggregate(y3, src3, dst3)

    H2 = D // 2
    pW3p = jnp.zeros((H2, D), _F32).at[:, :2].set(p_W3)
    pb3p = jnp.zeros((1, D), _F32).at[:, :2].set(p_b3.reshape(1, 2))
    out = _tc_fin(
        dis, xw3, z3, c3_b.reshape(1, D), h2,
        r_W1, r_b1.reshape(1, H2), r_g1.reshape(1, H2), r_be1.reshape(1, H2),
        r_rm1.reshape(1, H2), r_rv1.reshape(1, H2),
        r_W2.reshape(1, H2), r_b2.reshape(1, 1),
        p_W1, p_b1.reshape(1, D), p_g1.reshape(1, D), p_be1.reshape(1, D),
        p_rm1.reshape(1, D), p_rv1.reshape(1, D),
        p_W2, p_b2.reshape(1, H2), p_g2.reshape(1, H2), p_be2.reshape(1, H2),
        p_rm2.reshape(1, H2), p_rv2.reshape(1, H2),
        pW3p, pb3p)
    return out[:, :2]
